# baseline (speedup 1.0000x reference)
import functools

import jax
import jax.numpy as jnp
from jax.experimental import pallas as pl
from jax.experimental.pallas import tpu as pltpu

_SPLIT = 4
_SUB_ROWS = 1024
_BLOCK_ROWS = _SPLIT * _SUB_ROWS


def _rowdot_sigmoid_body(*refs):
    x_refs = refs[:_SPLIT]
    w_ref, b_ref, o_ref = refs[_SPLIT:]
    w = w_ref[...]
    hb = 0.5 * b_ref[0, 0]
    for j, x_ref in enumerate(x_refs):
        s = jnp.sum(x_ref[...] * w, axis=1, keepdims=True)
        h = 0.5 * s.T + hb
        o_ref[:, j * _SUB_ROWS:(j + 1) * _SUB_ROWS] = 0.5 * jnp.tanh(h) + 0.5


@jax.jit
def _logreg_sigmoid(x, weight, bias):
    n, f = x.shape
    tm = min(_BLOCK_ROWS, n)
    grid = pl.cdiv(n, tm)
    bias2d = bias.reshape(1, 1).astype(jnp.float32)

    x_block_bytes = tm * f * jnp.dtype(x.dtype).itemsize
    vmem_limit = int(min(2 * x_block_bytes + (4 << 20), 60 << 20))

    def _sub_spec(j):
        return pl.BlockSpec((_SUB_ROWS, f), lambda i, j=j: (_SPLIT * i + j, 0))

    out = pl.pallas_call(
        _rowdot_sigmoid_body,
        out_shape=jax.ShapeDtypeStruct((1, n), x.dtype),
        grid=(grid,),
        in_specs=[_sub_spec(j) for j in range(_SPLIT)] + [
            pl.BlockSpec((1, f), lambda i: (0, 0)),
            pl.BlockSpec((1, 1), lambda i: (0, 0), memory_space=pltpu.SMEM),
        ],
        out_specs=pl.BlockSpec((1, tm), lambda i: (0, i)),
        compiler_params=pltpu.CompilerParams(
            dimension_semantics=("parallel",),
            vmem_limit_bytes=vmem_limit,
        ),
    )(*([x] * _SPLIT), weight, bias2d)
    return out.reshape(n, 1)


def kernel(x, weight, bias):
    return _logreg_sigmoid(x, weight, bias)


# speedup vs baseline: 1.0164x; 1.0164x over previous
import functools

import jax
import jax.numpy as jnp
from jax.experimental import pallas as pl
from jax.experimental.pallas import tpu as pltpu

_SPLIT = 2
_SUB_ROWS = 1024
_BLOCK_ROWS = _SPLIT * _SUB_ROWS


def _rowdot_sigmoid_body(*refs):
    x_refs = refs[:_SPLIT]
    w_ref, b_ref, o_ref = refs[_SPLIT:]
    w = w_ref[...]
    hb = 0.5 * b_ref[0, 0]
    for j, x_ref in enumerate(x_refs):
        s = jnp.sum(x_ref[...] * w, axis=1, keepdims=True)
        h = 0.5 * s.T + hb
        o_ref[:, j * _SUB_ROWS:(j + 1) * _SUB_ROWS] = 0.5 * jnp.tanh(h) + 0.5


@jax.jit
def _logreg_sigmoid(x, weight, bias):
    n, f = x.shape
    tm = min(_BLOCK_ROWS, n)
    grid = pl.cdiv(n, tm)
    bias2d = bias.reshape(1, 1).astype(jnp.float32)

    x_block_bytes = tm * f * jnp.dtype(x.dtype).itemsize
    vmem_limit = int(min(2 * x_block_bytes + (4 << 20), 60 << 20))

    def _sub_spec(j):
        return pl.BlockSpec((_SUB_ROWS, f), lambda i, j=j: (_SPLIT * i + j, 0))

    out = pl.pallas_call(
        _rowdot_sigmoid_body,
        out_shape=jax.ShapeDtypeStruct((1, n), x.dtype),
        grid=(grid,),
        in_specs=[_sub_spec(j) for j in range(_SPLIT)] + [
            pl.BlockSpec((1, f), lambda i: (0, 0)),
            pl.BlockSpec((1, 1), lambda i: (0, 0), memory_space=pltpu.SMEM),
        ],
        out_specs=pl.BlockSpec((1, tm), lambda i: (0, i)),
        compiler_params=pltpu.CompilerParams(
            dimension_semantics=("parallel",),
            vmem_limit_bytes=vmem_limit,
        ),
    )(*([x] * _SPLIT), weight, bias2d)
    return out.reshape(n, 1)


def kernel(x, weight, bias):
    return _logreg_sigmoid(x, weight, bias)
